# baseline (speedup 1.0000x reference)
import functools

import jax
import jax.numpy as jnp
from jax import lax
from jax.experimental import pallas as pl
from jax.experimental.pallas import tpu as pltpu
from jax.experimental.pallas import tpu_sc as plsc

F = 26
VOCAB = 100000
V1 = VOCAB + 1
EMB = 16
N = 16384
EPS = 1e-5
NC = 2
NS = 16
NW = NC * NS
CH = 128
NCH = N // (NW * CH)
NG = CH // EMB

_mesh = plsc.VectorSubcoreMesh(core_axis_name="c", subcore_axis_name="s")


@functools.partial(
    pl.kernel,
    out_type=jax.ShapeDtypeStruct((N,), jnp.float32),
    mesh=_mesh,
    compiler_params=pltpu.CompilerParams(
        needs_layout_passes=False, use_tc_tiling_on_sc=False),
    scratch_types=[
        pltpu.VMEM((CH, 2 * EMB), jnp.int32),
        pltpu.VMEM((CH, 2 * EMB), jnp.float32),
        pltpu.VMEM((F, CH), jnp.int32),
        pltpu.VMEM((F, CH), jnp.float32),
        pltpu.VMEM((F, EMB, CH), jnp.float32),
        pltpu.VMEM((F, CH), jnp.float32),
        pltpu.VMEM((F, EMB), jnp.float32),
        pltpu.VMEM((EMB,), jnp.float32),
        pltpu.VMEM((CH,), jnp.float32),
        pltpu.SemaphoreType.DMA,
        pltpu.SemaphoreType.DMA,
    ],
)
def _deepfm_sc(t2, t1, xip, xvp, vseg, cvec, out,
               iblk_v, xblk_v, idx_v, xv_v, g2_v, g1_v, vseg_v, cv_v, out_v,
               semA, semB):
    wid = lax.axis_index("s") * NC + lax.axis_index("c")
    pltpu.sync_copy(vseg, vseg_v)
    pltpu.sync_copy(cvec, cv_v)
    lane = jnp.arange(EMB, dtype=jnp.int32)

    for ch in range(NCH):
        base = wid * (NCH * CH) + ch * CH
        pltpu.sync_copy(xip.at[pl.ds(base, CH)], iblk_v)
        pltpu.sync_copy(xvp.at[pl.ds(base, CH)], xblk_v)

        def _tr(f, _):
            fvec = jnp.full((EMB,), f, jnp.int32)
            for g in range(NG):
                rows = lane + g * EMB
                idx_v[f, pl.ds(g * EMB, EMB)] = plsc.load_gather(
                    iblk_v, [rows, fvec])
                xv_v[f, pl.ds(g * EMB, EMB)] = plsc.load_gather(
                    xblk_v, [rows, fvec])
            return 0

        lax.fori_loop(0, F, _tr, 0)

        def _issue(f, _):
            idxs = idx_v.at[f]

            def _issue_d(d, _):
                pltpu.make_async_copy(
                    t2.at[f, d].at[idxs], g2_v.at[f, d], semA).start()
                return 0

            lax.fori_loop(0, EMB, _issue_d, 0)
            pltpu.make_async_copy(t1.at[f].at[idxs], g1_v.at[f], semB).start()
            return 0

        lax.fori_loop(0, F, _issue, 0)

        def _drain(f, _):
            def _drain_d(d, _):
                pltpu.make_async_copy(
                    t2.at[0, 0].at[idx_v.at[0]], g2_v.at[0, 0], semA).wait()
                return 0

            lax.fori_loop(0, EMB, _drain_d, 0)
            pltpu.make_async_copy(
                t1.at[0].at[idx_v.at[0]], g1_v.at[0], semB).wait()
            return 0

        lax.fori_loop(0, F, _drain, 0)

        def _group(gi, _):
            sl = pl.ds(gi * EMB, EMB)
            xvs = [xv_v[f, sl] for f in range(F)]
            vrows = [vseg_v[f] for f in range(F)]
            tot = cv_v[...]
            for f in range(F):
                tot = tot + g1_v[f, sl] * xvs[f]
            for d in range(EMB):
                S = jnp.zeros((EMB,), jnp.float32)
                Q = jnp.zeros((EMB,), jnp.float32)
                for f in range(F):
                    fv = g2_v[f, d, sl] * xvs[f]
                    S = S + fv
                    Q = Q + fv * fv
                    tot = tot + fv * vrows[f][d]
                tot = tot + (S * S - Q) * 0.5
            out_v[sl] = tot
            return 0

        lax.fori_loop(0, NG, _group, 0)

        pltpu.sync_copy(out_v, out.at[pl.ds(base, CH)])


def kernel(xi, xv, emb1, emb2, W1, b1, gamma1, beta1, W2, b2, gamma2, beta2, bias):
    s = jnp.sqrt(jnp.float32(1.0 + EPS))
    g1s = gamma1 / s
    g2s = gamma2 / s
    u = W2.T @ g2s
    v = W1.T @ (g1s * u)
    c = jnp.dot(b1, g1s * u) + jnp.dot(beta1, u) + jnp.sum(g2s * b2 + beta2)
    const = c + bias[0]

    idx = xi[:, :, 0].astype(jnp.int32)
    xip = jnp.pad(idx, ((0, 0), (0, 2 * EMB - F)))
    xvp = jnp.pad(xv, ((0, 0), (0, 2 * EMB - F)))
    t2 = jnp.transpose(emb2, (0, 2, 1))
    t1 = emb1[:, :, 0]
    vseg = v.reshape(F, EMB).astype(jnp.float32)
    cvec = jnp.full((EMB,), const, dtype=jnp.float32)
    return _deepfm_sc(t2, t1, xip, xvp, vseg, cvec)


# speedup vs baseline: 1.3952x; 1.3952x over previous
import functools

import jax
import jax.numpy as jnp
from jax import lax
from jax.experimental import pallas as pl
from jax.experimental.pallas import tpu as pltpu
from jax.experimental.pallas import tpu_sc as plsc

F = 26
VOCAB = 100000
V1 = VOCAB + 1
EMB = 16
N = 16384
EPS = 1e-5
NC = 2
NS = 16
NW = NC * NS
CH = 128
NCH = N // (NW * CH)
NG = CH // EMB

VB = 784
VROW = VB * 128

_mesh = plsc.VectorSubcoreMesh(core_axis_name="c", subcore_axis_name="s")


def _detile_body(i_ref, o_ref):
    o_ref[...] = i_ref[0].reshape(EMB, 8, 128)


_detile = pl.pallas_call(
    _detile_body,
    grid=(F, VB // 8),
    in_specs=[pl.BlockSpec((1, EMB, 1024), lambda f, j: (f, 0, j))],
    out_specs=pl.BlockSpec((EMB, 8, 128), lambda f, j: (f, j, 0)),
    out_shape=jax.ShapeDtypeStruct((F * EMB, VB, 128), jnp.float32),
)


@functools.partial(
    pl.kernel,
    out_type=jax.ShapeDtypeStruct((N,), jnp.float32),
    mesh=_mesh,
    compiler_params=pltpu.CompilerParams(
        needs_layout_passes=False, use_tc_tiling_on_sc=False),
    scratch_types=[
        pltpu.VMEM((CH, 2 * EMB), jnp.int32),
        pltpu.VMEM((CH, 2 * EMB), jnp.float32),
        pltpu.VMEM((F, CH), jnp.int32),
        pltpu.VMEM((F, CH), jnp.float32),
        pltpu.VMEM((F, EMB, CH), jnp.float32),
        pltpu.VMEM((F, CH), jnp.float32),
        pltpu.VMEM((F, EMB), jnp.float32),
        pltpu.VMEM((EMB,), jnp.float32),
        pltpu.VMEM((CH,), jnp.float32),
        pltpu.SemaphoreType.DMA,
        pltpu.SemaphoreType.DMA,
    ],
)
def _deepfm_sc(t2, t1, xip, xvp, vseg, cvec, out,
               iblk_v, xblk_v, idx_v, xv_v, g2_v, g1_v, vseg_v, cv_v, out_v,
               semA, semB):
    wid = lax.axis_index("s") * NC + lax.axis_index("c")
    pltpu.sync_copy(vseg, vseg_v)
    pltpu.sync_copy(cvec, cv_v)
    lane = jnp.arange(EMB, dtype=jnp.int32)

    for ch in range(NCH):
        base = wid * (NCH * CH) + ch * CH
        pltpu.sync_copy(xip.at[pl.ds(base, CH)], iblk_v)
        pltpu.sync_copy(xvp.at[pl.ds(base, CH)], xblk_v)

        def _tr(f, _):
            fvec = jnp.full((EMB,), f, jnp.int32)
            for g in range(NG):
                rows = lane + g * EMB
                idx_v[f, pl.ds(g * EMB, EMB)] = plsc.load_gather(
                    iblk_v, [rows, fvec])
                xv_v[f, pl.ds(g * EMB, EMB)] = plsc.load_gather(
                    xblk_v, [rows, fvec])
            return 0

        lax.fori_loop(0, F, _tr, 0)

        def _issue(f, _):
            idxs = idx_v.at[f]

            def _issue_d(d, _):
                row = (f * EMB + d) * VROW
                pltpu.make_async_copy(
                    t2.at[pl.ds(row, VROW)].at[idxs], g2_v.at[f, d],
                    semA).start()
                return 0

            lax.fori_loop(0, EMB, _issue_d, 0)
            pltpu.make_async_copy(t1.at[f].at[idxs], g1_v.at[f], semB).start()
            return 0

        lax.fori_loop(0, F, _issue, 0)

        def _drain(f, _):
            def _drain_d(d, _):
                pltpu.make_async_copy(
                    t2.at[pl.ds(0, VROW)].at[idx_v.at[0]], g2_v.at[0, 0],
                    semA).wait()
                return 0

            lax.fori_loop(0, EMB, _drain_d, 0)
            pltpu.make_async_copy(
                t1.at[0].at[idx_v.at[0]], g1_v.at[0], semB).wait()
            return 0

        lax.fori_loop(0, F, _drain, 0)

        def _group(gi, _):
            sl = pl.ds(gi * EMB, EMB)
            xvs = [xv_v[f, sl] for f in range(F)]
            vrows = [vseg_v[f] for f in range(F)]
            tot = cv_v[...]
            for f in range(F):
                tot = tot + g1_v[f, sl] * xvs[f]
            for d in range(EMB):
                S = jnp.zeros((EMB,), jnp.float32)
                Q = jnp.zeros((EMB,), jnp.float32)
                for f in range(F):
                    fv = g2_v[f, d, sl] * xvs[f]
                    S = S + fv
                    Q = Q + fv * fv
                    tot = tot + fv * vrows[f][d]
                tot = tot + (S * S - Q) * 0.5
            out_v[sl] = tot
            return 0

        lax.fori_loop(0, NG, _group, 0)

        pltpu.sync_copy(out_v, out.at[pl.ds(base, CH)])


def kernel(xi, xv, emb1, emb2, W1, b1, gamma1, beta1, W2, b2, gamma2, beta2, bias):
    s = jnp.sqrt(jnp.float32(1.0 + EPS))
    g1s = gamma1 / s
    g2s = gamma2 / s
    u = W2.T @ g2s
    v = W1.T @ (g1s * u)
    c = jnp.dot(b1, g1s * u) + jnp.dot(beta1, u) + jnp.sum(g2s * b2 + beta2)
    const = c + bias[0]

    idx = xi[:, :, 0].astype(jnp.int32)
    xip = jnp.pad(idx, ((0, 0), (0, 2 * EMB - F)))
    xvp = jnp.pad(xv, ((0, 0), (0, 2 * EMB - F)))
    t2t = jnp.transpose(emb2, (0, 2, 1))
    t2 = _detile(t2t).reshape(F * EMB * VROW)
    t1 = emb1[:, :, 0]
    vseg = v.reshape(F, EMB).astype(jnp.float32)
    cvec = jnp.full((EMB,), const, dtype=jnp.float32)
    return _deepfm_sc(t2, t1, xip, xvp, vseg, cvec)


# speedup vs baseline: 1.4083x; 1.0094x over previous
import functools

import jax
import jax.numpy as jnp
from jax import lax
from jax.experimental import pallas as pl
from jax.experimental.pallas import tpu as pltpu
from jax.experimental.pallas import tpu_sc as plsc

F = 26
VOCAB = 100000
V1 = VOCAB + 1
EMB = 16
N = 16384
EPS = 1e-5
NC = 2
NS = 16
NW = NC * NS
CH = 128
NCH = N // (NW * CH)
NG = CH // EMB

VB = 784
FSTR = VB * EMB * 128
GLEN = ((VOCAB >> 7) * 2048 + 128 + 7) // 8 * 8

_mesh = plsc.VectorSubcoreMesh(core_axis_name="c", subcore_axis_name="s")


def _detile_body(i_ref, o_ref):
    for vb in range(8):
        o_ref[0, vb] = i_ref[0, :, 128 * vb:128 * (vb + 1)]


_detile = pl.pallas_call(
    _detile_body,
    grid=(F, VB // 8),
    in_specs=[pl.BlockSpec((1, EMB, 1024), lambda f, j: (f, 0, j))],
    out_specs=pl.BlockSpec((1, 8, EMB, 128), lambda f, j: (f, j, 0, 0)),
    out_shape=jax.ShapeDtypeStruct((F, VB, EMB, 128), jnp.float32),
)


@functools.partial(
    pl.kernel,
    out_type=jax.ShapeDtypeStruct((N,), jnp.float32),
    mesh=_mesh,
    compiler_params=pltpu.CompilerParams(
        needs_layout_passes=False, use_tc_tiling_on_sc=False),
    scratch_types=[
        pltpu.VMEM((CH, 2 * EMB), jnp.int32),
        pltpu.VMEM((CH, 2 * EMB), jnp.float32),
        pltpu.VMEM((F, CH), jnp.int32),
        pltpu.VMEM((F, CH), jnp.int32),
        pltpu.VMEM((F, CH), jnp.float32),
        pltpu.VMEM((F, EMB, CH), jnp.float32),
        pltpu.VMEM((F, CH), jnp.float32),
        pltpu.VMEM((F, EMB), jnp.float32),
        pltpu.VMEM((EMB,), jnp.float32),
        pltpu.VMEM((CH,), jnp.float32),
        pltpu.SemaphoreType.DMA,
        pltpu.SemaphoreType.DMA,
    ],
)
def _deepfm_sc(t2, t1, xip, xvp, vseg, cvec, out,
               iblk_v, xblk_v, idx_v, idx1_v, xv_v, g2_v, g1_v, vseg_v, cv_v,
               out_v, semA, semB):
    wid = lax.axis_index("s") * NC + lax.axis_index("c")
    pltpu.sync_copy(vseg, vseg_v)
    pltpu.sync_copy(cvec, cv_v)
    lane = jnp.arange(EMB, dtype=jnp.int32)

    for ch in range(NCH):
        base = wid * (NCH * CH) + ch * CH
        pltpu.sync_copy(xip.at[pl.ds(base, CH)], iblk_v)
        pltpu.sync_copy(xvp.at[pl.ds(base, CH)], xblk_v)

        def _tr(f, _):
            fvec = jnp.full((EMB,), f, jnp.int32)
            for g in range(NG):
                rows = lane + g * EMB
                vals = plsc.load_gather(iblk_v, [rows, fvec])
                idx1_v[f, pl.ds(g * EMB, EMB)] = vals
                idx_v[f, pl.ds(g * EMB, EMB)] = (
                    (vals >> 7) * 2048 + (vals & 127))
                xv_v[f, pl.ds(g * EMB, EMB)] = plsc.load_gather(
                    xblk_v, [rows, fvec])
            return 0

        lax.fori_loop(0, F, _tr, 0)

        def _issue(f, _):
            idxs = idx_v.at[f]

            def _issue_d(d, _):
                pltpu.make_async_copy(
                    t2.at[pl.ds(f * FSTR + d * 128, GLEN)].at[idxs],
                    g2_v.at[f, d], semA).start()
                return 0

            lax.fori_loop(0, EMB, _issue_d, 0)
            pltpu.make_async_copy(
                t1.at[f].at[idx1_v.at[f]], g1_v.at[f], semB).start()
            return 0

        lax.fori_loop(0, F, _issue, 0)

        def _drain(f, _):
            def _drain_d(d, _):
                pltpu.make_async_copy(
                    t2.at[pl.ds(0, GLEN)].at[idx_v.at[0]], g2_v.at[0, 0],
                    semA).wait()
                return 0

            lax.fori_loop(0, EMB, _drain_d, 0)
            pltpu.make_async_copy(
                t1.at[0].at[idx1_v.at[0]], g1_v.at[0], semB).wait()
            return 0

        lax.fori_loop(0, F, _drain, 0)

        def _group(gi, _):
            sl = pl.ds(gi * EMB, EMB)
            xvs = [xv_v[f, sl] for f in range(F)]
            vrows = [vseg_v[f] for f in range(F)]
            tot = cv_v[...]
            for f in range(F):
                tot = tot + g1_v[f, sl] * xvs[f]
            for d in range(EMB):
                S = jnp.zeros((EMB,), jnp.float32)
                Q = jnp.zeros((EMB,), jnp.float32)
                for f in range(F):
                    fv = g2_v[f, d, sl] * xvs[f]
                    S = S + fv
                    Q = Q + fv * fv
                    tot = tot + fv * vrows[f][d]
                tot = tot + (S * S - Q) * 0.5
            out_v[sl] = tot
            return 0

        lax.fori_loop(0, NG, _group, 0)

        pltpu.sync_copy(out_v, out.at[pl.ds(base, CH)])


def kernel(xi, xv, emb1, emb2, W1, b1, gamma1, beta1, W2, b2, gamma2, beta2, bias):
    s = jnp.sqrt(jnp.float32(1.0 + EPS))
    g1s = gamma1 / s
    g2s = gamma2 / s
    u = W2.T @ g2s
    v = W1.T @ (g1s * u)
    c = jnp.dot(b1, g1s * u) + jnp.dot(beta1, u) + jnp.sum(g2s * b2 + beta2)
    const = c + bias[0]

    idx = xi[:, :, 0].astype(jnp.int32)
    xip = jnp.pad(idx, ((0, 0), (0, 2 * EMB - F)))
    xvp = jnp.pad(xv, ((0, 0), (0, 2 * EMB - F)))
    t2t = jnp.transpose(emb2, (0, 2, 1))
    t2 = _detile(t2t).reshape(F * FSTR)
    t1 = emb1[:, :, 0]
    vseg = v.reshape(F, EMB).astype(jnp.float32)
    cvec = jnp.full((EMB,), const, dtype=jnp.float32)
    return _deepfm_sc(t2, t1, xip, xvp, vseg, cvec)


# speedup vs baseline: 3.6461x; 2.5891x over previous
import functools

import jax
import jax.numpy as jnp
from jax import lax
from jax.experimental import pallas as pl
from jax.experimental.pallas import tpu as pltpu
from jax.experimental.pallas import tpu_sc as plsc

F = 26
VOCAB = 100000
V1 = VOCAB + 1
EMB = 16
N = 16384
EPS = 1e-5
NC = 2
NS = 16
NW = NC * NS
CH = 128
NCH = N // (NW * CH)
NG = CH // EMB

VB = 784
FSTR = VB * EMB * 128
GLEN = ((VOCAB >> 7) * 2048 + 128 + 7) // 8 * 8

_mesh = plsc.VectorSubcoreMesh(core_axis_name="c", subcore_axis_name="s")


VBBLK = 56


def _detile_body(i_ref, o_ref):
    for vb in range(VBBLK):
        o_ref[0, vb] = i_ref[0, :, 128 * vb:128 * (vb + 1)]


_detile = pl.pallas_call(
    _detile_body,
    grid=(F, VB // VBBLK),
    in_specs=[pl.BlockSpec((1, EMB, VBBLK * 128), lambda f, j: (f, 0, j))],
    out_specs=pl.BlockSpec((1, VBBLK, EMB, 128), lambda f, j: (f, j, 0, 0)),
    out_shape=jax.ShapeDtypeStruct((F, VB, EMB, 128), jnp.float32),
)


@functools.partial(
    pl.kernel,
    out_type=jax.ShapeDtypeStruct((N,), jnp.float32),
    mesh=_mesh,
    compiler_params=pltpu.CompilerParams(
        needs_layout_passes=False, use_tc_tiling_on_sc=False),
    scratch_types=[
        pltpu.VMEM((CH, 2 * EMB), jnp.int32),
        pltpu.VMEM((CH, 2 * EMB), jnp.float32),
        pltpu.VMEM((F, CH), jnp.int32),
        pltpu.VMEM((F, CH), jnp.int32),
        pltpu.VMEM((F, CH), jnp.float32),
        pltpu.VMEM((F, EMB, CH), jnp.float32),
        pltpu.VMEM((F, CH), jnp.float32),
        pltpu.VMEM((F, EMB), jnp.float32),
        pltpu.VMEM((EMB,), jnp.float32),
        pltpu.VMEM((CH,), jnp.float32),
        pltpu.SemaphoreType.DMA,
        pltpu.SemaphoreType.DMA,
    ],
)
def _deepfm_sc(t2, t1, xip, xvp, vseg, cvec, out,
               iblk_v, xblk_v, idx_v, idx1_v, xv_v, g2_v, g1_v, vseg_v, cv_v,
               out_v, semA, semB):
    wid = lax.axis_index("s") * NC + lax.axis_index("c")
    pltpu.sync_copy(vseg, vseg_v)
    pltpu.sync_copy(cvec, cv_v)
    lane = jnp.arange(EMB, dtype=jnp.int32)

    for ch in range(NCH):
        base = wid * (NCH * CH) + ch * CH
        pltpu.sync_copy(xip.at[pl.ds(base, CH)], iblk_v)
        pltpu.sync_copy(xvp.at[pl.ds(base, CH)], xblk_v)

        def _tr(f, _):
            fvec = jnp.full((EMB,), f, jnp.int32)
            for g in range(NG):
                rows = lane + g * EMB
                vals = plsc.load_gather(iblk_v, [rows, fvec])
                idx1_v[f, pl.ds(g * EMB, EMB)] = vals
                idx_v[f, pl.ds(g * EMB, EMB)] = (
                    (vals >> 7) * 2048 + (vals & 127))
                xv_v[f, pl.ds(g * EMB, EMB)] = plsc.load_gather(
                    xblk_v, [rows, fvec])
            return 0

        lax.fori_loop(0, F, _tr, 0)

        def _issue(f, _):
            idxs = idx_v.at[f]

            def _issue_d(d, _):
                pltpu.make_async_copy(
                    t2.at[pl.ds(f * FSTR + d * 128, GLEN)].at[idxs],
                    g2_v.at[f, d], semA).start()
                return 0

            lax.fori_loop(0, EMB, _issue_d, 0)
            pltpu.make_async_copy(
                t1.at[f].at[idx1_v.at[f]], g1_v.at[f], semB).start()
            return 0

        lax.fori_loop(0, F, _issue, 0)

        def _drain(f, _):
            def _drain_d(d, _):
                pltpu.make_async_copy(
                    t2.at[pl.ds(0, GLEN)].at[idx_v.at[0]], g2_v.at[0, 0],
                    semA).wait()
                return 0

            lax.fori_loop(0, EMB, _drain_d, 0)
            pltpu.make_async_copy(
                t1.at[0].at[idx1_v.at[0]], g1_v.at[0], semB).wait()
            return 0

        lax.fori_loop(0, F, _drain, 0)

        def _group(gi, _):
            sl = pl.ds(gi * EMB, EMB)
            xvs = [xv_v[f, sl] for f in range(F)]
            vrows = [vseg_v[f] for f in range(F)]
            tot = cv_v[...]
            for f in range(F):
                tot = tot + g1_v[f, sl] * xvs[f]
            for d in range(EMB):
                S = jnp.zeros((EMB,), jnp.float32)
                Q = jnp.zeros((EMB,), jnp.float32)
                for f in range(F):
                    fv = g2_v[f, d, sl] * xvs[f]
                    S = S + fv
                    Q = Q + fv * fv
                    tot = tot + fv * vrows[f][d]
                tot = tot + (S * S - Q) * 0.5
            out_v[sl] = tot
            return 0

        lax.fori_loop(0, NG, _group, 0)

        pltpu.sync_copy(out_v, out.at[pl.ds(base, CH)])


def kernel(xi, xv, emb1, emb2, W1, b1, gamma1, beta1, W2, b2, gamma2, beta2, bias):
    s = jnp.sqrt(jnp.float32(1.0 + EPS))
    g1s = gamma1 / s
    g2s = gamma2 / s
    u = W2.T @ g2s
    v = W1.T @ (g1s * u)
    c = jnp.dot(b1, g1s * u) + jnp.dot(beta1, u) + jnp.sum(g2s * b2 + beta2)
    const = c + bias[0]

    idx = xi[:, :, 0].astype(jnp.int32)
    xip = jnp.pad(idx, ((0, 0), (0, 2 * EMB - F)))
    xvp = jnp.pad(xv, ((0, 0), (0, 2 * EMB - F)))
    t2t = jnp.transpose(emb2, (0, 2, 1))
    t2 = _detile(t2t).reshape(F * FSTR)
    t1 = emb1[:, :, 0]
    vseg = v.reshape(F, EMB).astype(jnp.float32)
    cvec = jnp.full((EMB,), const, dtype=jnp.float32)
    return _deepfm_sc(t2, t1, xip, xvp, vseg, cvec)


# speedup vs baseline: 4.1655x; 1.1424x over previous
import functools

import jax
import jax.numpy as jnp
from jax import lax
from jax.experimental import pallas as pl
from jax.experimental.pallas import tpu as pltpu
from jax.experimental.pallas import tpu_sc as plsc

F = 26
VOCAB = 100000
V1 = VOCAB + 1
EMB = 16
N = 16384
EPS = 1e-5
NC = 2
NS = 16
NW = NC * NS
CH = 128
NCH = N // (NW * CH)
NG = CH // EMB

VB = 784
FSTR = VB * EMB * 128
GLEN = ((VOCAB >> 7) * 2048 + 128 + 7) // 8 * 8

_mesh = plsc.VectorSubcoreMesh(core_axis_name="c", subcore_axis_name="s")


VBBLK = 112


def _detile_body(i_ref, o_ref):
    for vb in range(VBBLK):
        o_ref[0, vb] = i_ref[0, :, 128 * vb:128 * (vb + 1)]


_detile = pl.pallas_call(
    _detile_body,
    grid=(F, VB // VBBLK),
    in_specs=[pl.BlockSpec((1, EMB, VBBLK * 128), lambda f, j: (f, 0, j))],
    out_specs=pl.BlockSpec((1, VBBLK, EMB, 128), lambda f, j: (f, j, 0, 0)),
    out_shape=jax.ShapeDtypeStruct((F, VB, EMB, 128), jnp.float32),
)


@functools.partial(
    pl.kernel,
    out_type=jax.ShapeDtypeStruct((N,), jnp.float32),
    mesh=_mesh,
    compiler_params=pltpu.CompilerParams(
        needs_layout_passes=False, use_tc_tiling_on_sc=False),
    scratch_types=[
        pltpu.VMEM((CH, 2 * EMB), jnp.int32),
        pltpu.VMEM((CH, 2 * EMB), jnp.float32),
        pltpu.VMEM((F, CH), jnp.int32),
        pltpu.VMEM((F, CH), jnp.int32),
        pltpu.VMEM((F, CH), jnp.float32),
        pltpu.VMEM((F, EMB, CH), jnp.float32),
        pltpu.VMEM((F, CH), jnp.float32),
        pltpu.VMEM((F, EMB), jnp.float32),
        pltpu.VMEM((EMB,), jnp.float32),
        pltpu.VMEM((CH,), jnp.float32),
        pltpu.SemaphoreType.DMA,
        pltpu.SemaphoreType.DMA,
    ],
)
def _deepfm_sc(t2, t1, xip, xvp, vseg, cvec, out,
               iblk_v, xblk_v, idx_v, idx1_v, xv_v, g2_v, g1_v, vseg_v, cv_v,
               out_v, semA, semB):
    wid = lax.axis_index("s") * NC + lax.axis_index("c")
    pltpu.sync_copy(vseg, vseg_v)
    pltpu.sync_copy(cvec, cv_v)
    lane = jnp.arange(EMB, dtype=jnp.int32)

    for ch in range(NCH):
        base = wid * (NCH * CH) + ch * CH
        pltpu.sync_copy(xip.at[pl.ds(base, CH)], iblk_v)
        pltpu.sync_copy(xvp.at[pl.ds(base, CH)], xblk_v)

        def _tr(f, _):
            fvec = jnp.full((EMB,), f, jnp.int32)
            for g in range(NG):
                rows = lane + g * EMB
                vals = plsc.load_gather(iblk_v, [rows, fvec])
                idx1_v[f, pl.ds(g * EMB, EMB)] = vals
                idx_v[f, pl.ds(g * EMB, EMB)] = (
                    (vals >> 7) * 2048 + (vals & 127))
                xv_v[f, pl.ds(g * EMB, EMB)] = plsc.load_gather(
                    xblk_v, [rows, fvec])
            return 0

        lax.fori_loop(0, F, _tr, 0)

        def _issue(f, _):
            idxs = idx_v.at[f]

            def _issue_d(d, _):
                pltpu.make_async_copy(
                    t2.at[pl.ds(f * FSTR + d * 128, GLEN)].at[idxs],
                    g2_v.at[f, d], semA).start()
                return 0

            lax.fori_loop(0, EMB, _issue_d, 0)
            pltpu.make_async_copy(
                t1.at[f].at[idx1_v.at[f]], g1_v.at[f], semB).start()
            return 0

        lax.fori_loop(0, F, _issue, 0)

        def _drain(f, _):
            def _drain_d(d, _):
                pltpu.make_async_copy(
                    t2.at[pl.ds(0, GLEN)].at[idx_v.at[0]], g2_v.at[0, 0],
                    semA).wait()
                return 0

            lax.fori_loop(0, EMB, _drain_d, 0)
            pltpu.make_async_copy(
                t1.at[0].at[idx1_v.at[0]], g1_v.at[0], semB).wait()
            return 0

        lax.fori_loop(0, F, _drain, 0)

        def _group(gi, _):
            sl = pl.ds(gi * EMB, EMB)
            xvs = [xv_v[f, sl] for f in range(F)]
            vrows = [vseg_v[f] for f in range(F)]
            tot = cv_v[...]
            for f in range(F):
                tot = tot + g1_v[f, sl] * xvs[f]
            for d in range(EMB):
                S = jnp.zeros((EMB,), jnp.float32)
                Q = jnp.zeros((EMB,), jnp.float32)
                for f in range(F):
                    fv = g2_v[f, d, sl] * xvs[f]
                    S = S + fv
                    Q = Q + fv * fv
                    tot = tot + fv * vrows[f][d]
                tot = tot + (S * S - Q) * 0.5
            out_v[sl] = tot
            return 0

        lax.fori_loop(0, NG, _group, 0)

        pltpu.sync_copy(out_v, out.at[pl.ds(base, CH)])


def kernel(xi, xv, emb1, emb2, W1, b1, gamma1, beta1, W2, b2, gamma2, beta2, bias):
    s = jnp.sqrt(jnp.float32(1.0 + EPS))
    g1s = gamma1 / s
    g2s = gamma2 / s
    u = W2.T @ g2s
    v = W1.T @ (g1s * u)
    c = jnp.dot(b1, g1s * u) + jnp.dot(beta1, u) + jnp.sum(g2s * b2 + beta2)
    const = c + bias[0]

    idx = xi[:, :, 0].astype(jnp.int32)
    xip = jnp.pad(idx, ((0, 0), (0, 2 * EMB - F)))
    xvp = jnp.pad(xv, ((0, 0), (0, 2 * EMB - F)))
    t2t = jnp.transpose(emb2, (0, 2, 1))
    t2 = _detile(t2t).reshape(F * FSTR)
    t1 = emb1[:, :, 0]
    vseg = v.reshape(F, EMB).astype(jnp.float32)
    cvec = jnp.full((EMB,), const, dtype=jnp.float32)
    return _deepfm_sc(t2, t1, xip, xvp, vseg, cvec)


# speedup vs baseline: 4.5244x; 1.0862x over previous
import functools

import jax
import jax.numpy as jnp
from jax import lax
from jax.experimental import pallas as pl
from jax.experimental.pallas import tpu as pltpu
from jax.experimental.pallas import tpu_sc as plsc

F = 26
VOCAB = 100000
V1 = VOCAB + 1
EMB = 16
N = 16384
EPS = 1e-5
NC = 2
NS = 16
NW = NC * NS
CH = 128
NCH = N // (NW * CH)
NG = CH // EMB

VB = 784
FSTR = VB * EMB * 128
GLEN = ((VOCAB >> 7) * 2048 + 128 + 7) // 8 * 8

_mesh = plsc.VectorSubcoreMesh(core_axis_name="c", subcore_axis_name="s")


VBBLK = 196


def _detile_body(i_ref, o_ref):
    for vb in range(VBBLK):
        o_ref[0, vb] = i_ref[0, :, 128 * vb:128 * (vb + 1)]


_detile = pl.pallas_call(
    _detile_body,
    grid=(F, VB // VBBLK),
    in_specs=[pl.BlockSpec((1, EMB, VBBLK * 128), lambda f, j: (f, 0, j))],
    out_specs=pl.BlockSpec((1, VBBLK, EMB, 128), lambda f, j: (f, j, 0, 0)),
    out_shape=jax.ShapeDtypeStruct((F, VB, EMB, 128), jnp.float32),
)


@functools.partial(
    pl.kernel,
    out_type=jax.ShapeDtypeStruct((N,), jnp.float32),
    mesh=_mesh,
    compiler_params=pltpu.CompilerParams(
        needs_layout_passes=False, use_tc_tiling_on_sc=False),
    scratch_types=[
        pltpu.VMEM((CH, 2 * EMB), jnp.int32),
        pltpu.VMEM((CH, 2 * EMB), jnp.float32),
        pltpu.VMEM((F, CH), jnp.int32),
        pltpu.VMEM((F, CH), jnp.int32),
        pltpu.VMEM((F, CH), jnp.float32),
        pltpu.VMEM((F, EMB, CH), jnp.float32),
        pltpu.VMEM((F, CH), jnp.float32),
        pltpu.VMEM((F, EMB), jnp.float32),
        pltpu.VMEM((EMB,), jnp.float32),
        pltpu.VMEM((CH,), jnp.float32),
        pltpu.SemaphoreType.DMA,
        pltpu.SemaphoreType.DMA,
    ],
)
def _deepfm_sc(t2, t1, xip, xvp, vseg, cvec, out,
               iblk_v, xblk_v, idx_v, idx1_v, xv_v, g2_v, g1_v, vseg_v, cv_v,
               out_v, semA, semB):
    wid = lax.axis_index("s") * NC + lax.axis_index("c")
    pltpu.sync_copy(vseg, vseg_v)
    pltpu.sync_copy(cvec, cv_v)
    lane = jnp.arange(EMB, dtype=jnp.int32)

    for ch in range(NCH):
        base = wid * (NCH * CH) + ch * CH
        pltpu.sync_copy(xip.at[pl.ds(base, CH)], iblk_v)
        pltpu.sync_copy(xvp.at[pl.ds(base, CH)], xblk_v)

        def _tr(f, _):
            fvec = jnp.full((EMB,), f, jnp.int32)
            for g in range(NG):
                rows = lane + g * EMB
                vals = plsc.load_gather(iblk_v, [rows, fvec])
                idx1_v[f, pl.ds(g * EMB, EMB)] = vals
                idx_v[f, pl.ds(g * EMB, EMB)] = (
                    (vals >> 7) * 2048 + (vals & 127))
                xv_v[f, pl.ds(g * EMB, EMB)] = plsc.load_gather(
                    xblk_v, [rows, fvec])
            return 0

        lax.fori_loop(0, F, _tr, 0)

        def _issue(f, _):
            idxs = idx_v.at[f]

            def _issue_d(d, _):
                pltpu.make_async_copy(
                    t2.at[pl.ds(f * FSTR + d * 128, GLEN)].at[idxs],
                    g2_v.at[f, d], semA).start()
                return 0

            lax.fori_loop(0, EMB, _issue_d, 0)
            pltpu.make_async_copy(
                t1.at[f].at[idx1_v.at[f]], g1_v.at[f], semB).start()
            return 0

        lax.fori_loop(0, F, _issue, 0)

        def _drain(f, _):
            def _drain_d(d, _):
                pltpu.make_async_copy(
                    t2.at[pl.ds(0, GLEN)].at[idx_v.at[0]], g2_v.at[0, 0],
                    semA).wait()
                return 0

            lax.fori_loop(0, EMB, _drain_d, 0)
            pltpu.make_async_copy(
                t1.at[0].at[idx1_v.at[0]], g1_v.at[0], semB).wait()
            return 0

        lax.fori_loop(0, F, _drain, 0)

        def _group(gi, _):
            sl = pl.ds(gi * EMB, EMB)
            xvs = [xv_v[f, sl] for f in range(F)]
            vrows = [vseg_v[f] for f in range(F)]
            tot = cv_v[...]
            for f in range(F):
                tot = tot + g1_v[f, sl] * xvs[f]
            for d in range(EMB):
                S = jnp.zeros((EMB,), jnp.float32)
                Q = jnp.zeros((EMB,), jnp.float32)
                for f in range(F):
                    fv = g2_v[f, d, sl] * xvs[f]
                    S = S + fv
                    Q = Q + fv * fv
                    tot = tot + fv * vrows[f][d]
                tot = tot + (S * S - Q) * 0.5
            out_v[sl] = tot
            return 0

        lax.fori_loop(0, NG, _group, 0)

        pltpu.sync_copy(out_v, out.at[pl.ds(base, CH)])


def kernel(xi, xv, emb1, emb2, W1, b1, gamma1, beta1, W2, b2, gamma2, beta2, bias):
    s = jnp.sqrt(jnp.float32(1.0 + EPS))
    g1s = gamma1 / s
    g2s = gamma2 / s
    u = W2.T @ g2s
    v = W1.T @ (g1s * u)
    c = jnp.dot(b1, g1s * u) + jnp.dot(beta1, u) + jnp.sum(g2s * b2 + beta2)
    const = c + bias[0]

    idx = xi[:, :, 0].astype(jnp.int32)
    xip = jnp.pad(idx, ((0, 0), (0, 2 * EMB - F)))
    xvp = jnp.pad(xv, ((0, 0), (0, 2 * EMB - F)))
    t2t = jnp.transpose(emb2, (0, 2, 1))
    t2 = _detile(t2t).reshape(F * FSTR)
    t1 = emb1[:, :, 0]
    vseg = v.reshape(F, EMB).astype(jnp.float32)
    cvec = jnp.full((EMB,), const, dtype=jnp.float32)
    return _deepfm_sc(t2, t1, xip, xvp, vseg, cvec)
